# Initial kernel scaffold; baseline (speedup 1.0000x reference)
import jax
import jax.numpy as jnp
from jax.experimental import pallas as pl


def kernel(x, W_experts, b_experts, W_gate, b_gate):
    raise NotImplementedError("write your pallas kernel here")


# speedup vs baseline: 1.9283x; 1.9283x over previous
import jax
import jax.numpy as jnp
from jax.experimental import pallas as pl

H = 1024
E = 8
TM = 256


def _moe_body(xf_ref, xb_ref, wt_ref, br_ref, wg_ref, bg_ref, out_ref):
    xf = xf_ref[...]
    xb = xb_ref[...]

    logits = jax.lax.dot_general(
        xf, wg_ref[...], (((1,), (0,)), ((), ())),
        preferred_element_type=jnp.float32,
    ) + bg_ref[...]
    idx = jax.lax.broadcasted_iota(jnp.int32, (TM, E), 1)
    m1 = jnp.max(logits, axis=-1, keepdims=True)
    i1 = jnp.min(jnp.where(logits == m1, idx, E), axis=-1, keepdims=True)
    mask1 = idx == i1
    l2 = jnp.where(mask1, jnp.finfo(jnp.float32).min, logits)
    m2 = jnp.max(l2, axis=-1, keepdims=True)
    i2 = jnp.min(jnp.where(l2 == m2, idx, E), axis=-1, keepdims=True)
    mask2 = idx == i2
    t = jnp.exp(m2 - m1)
    w1 = 1.0 / (1.0 + t)
    w = jnp.where(mask1, w1, 0.0) + jnp.where(mask2, 1.0 - w1, 0.0)

    acc = jax.lax.dot_general(w, br_ref[...], (((1,), (0,)), ((), ())))

    for e in range(E):
        ye = jax.lax.dot_general(
            xb, wt_ref[e], (((1,), (0,)), ((), ())),
            preferred_element_type=jnp.float32,
        )
        acc = acc + w[:, e:e + 1] * ye
    out_ref[...] = acc


def kernel(x, W_experts, b_experts, W_gate, b_gate):
    B, S, _ = x.shape
    T = B * S
    xf = x.reshape(T, H)
    xb = xf.astype(jnp.bfloat16)
    wt = W_experts.reshape(H, E, H).transpose(1, 2, 0).astype(jnp.bfloat16)
    br = b_experts.reshape(H, E).T
    wg = W_gate.T
    bg = b_gate.reshape(1, E)

    out = pl.pallas_call(
        _moe_body,
        grid=(T // TM,),
        in_specs=[
            pl.BlockSpec((TM, H), lambda i: (i, 0)),
            pl.BlockSpec((TM, H), lambda i: (i, 0)),
            pl.BlockSpec((E, H, H), lambda i: (0, 0, 0)),
            pl.BlockSpec((E, H), lambda i: (0, 0)),
            pl.BlockSpec((H, E), lambda i: (0, 0)),
            pl.BlockSpec((1, E), lambda i: (0, 0)),
        ],
        out_specs=pl.BlockSpec((TM, H), lambda i: (i, 0)),
        out_shape=jax.ShapeDtypeStruct((T, H), jnp.float32),
    )(xf, xb, wt, br, wg, bg)
    return out.reshape(B, S, H)
